# Initial kernel scaffold; baseline (speedup 1.0000x reference)
import jax
import jax.numpy as jnp
from jax.experimental import pallas as pl


def kernel(x, edge_index, sectors, W0, b0, W1, b1, g0, be0, g1, be1, fcW1, fcb1, fcW2, fcb2, HW1, Hb1, HW2, Hb2):
    raise NotImplementedError("write your pallas kernel here")


# speedup vs baseline: 11.3561x; 11.3561x over previous
import functools

import jax
import jax.numpy as jnp
from jax import lax
from jax.experimental import pallas as pl
from jax.experimental.pallas import tpu as pltpu
from jax.experimental.pallas import tpu_sc as plsc

N = 10000
NP = 10240
E = 320000
EPAD = 327680
D_IN = 128
H = 256
HH = 128
S = 11
EPS = 1e-5
BNS = 1.0 / (1.0 + EPS) ** 0.5
K = 128
RB = NP // 16
R = 1024
NB = NP // R

_mesh = plsc.VectorSubcoreMesh(core_axis_name="c", subcore_axis_name="s")


@functools.partial(
    pl.kernel, mesh=_mesh,
    out_type=jax.ShapeDtypeStruct((2 * NP,), jnp.float32),
    scratch_types=[
        pltpu.VMEM((8, K), jnp.int32),
        pltpu.VMEM((K,), jnp.float32),
        pltpu.VMEM((RB,), jnp.float32),
        pltpu.VMEM_SHARED((NP,), jnp.float32),
    ],
)
def _deg(dst_hbm, out_hbm, idxbuf, ones_v, zbuf, acc):
    c = lax.axis_index("c")
    s = lax.axis_index("s")
    w = c * 16 + s

    def fill_ones(i, _):
        ones_v[pl.ds(i * 16, 16)] = jnp.ones((16,), jnp.float32)
        return 0

    lax.fori_loop(0, K // 16, fill_ones, 0)

    def fill_zero(i, _):
        zbuf[pl.ds(i * 16, 16)] = jnp.zeros((16,), jnp.float32)
        return 0

    lax.fori_loop(0, RB // 16, fill_zero, 0)
    pltpu.sync_copy(zbuf, acc.at[pl.ds(s * RB, RB)])
    plsc.subcore_barrier()

    e0 = w * (EPAD // 32)

    def chunk(g, _):
        pltpu.sync_copy(dst_hbm.at[pl.ds(e0 + g * K, K)], idxbuf.at[0])
        pltpu.sync_copy(ones_v, acc.at[idxbuf.at[0]], add=True)
        return 0

    lax.fori_loop(0, EPAD // 32 // K, chunk, 0)
    plsc.subcore_barrier()
    pltpu.sync_copy(acc.at[pl.ds(s * RB, RB)],
                    out_hbm.at[pl.ds(c * NP + s * RB, RB)])


@functools.partial(
    pl.kernel, mesh=_mesh,
    out_type=jax.ShapeDtypeStruct((2 * NP, HH), jnp.float32),
    scratch_types=[
        pltpu.VMEM((8, K), jnp.int32),
        pltpu.VMEM((8, K), jnp.int32),
        pltpu.VMEM((K, HH), jnp.float32),
        pltpu.VMEM_SHARED((NP, HH), jnp.float32),
    ],
)
def _agg(hs_hbm, srcs_hbm, dst_hbm, out_hbm, sbuf, dbuf, rows, acc):
    c = lax.axis_index("c")
    s = lax.axis_index("s")
    pltpu.sync_copy(hs_hbm.at[pl.ds(c * NP + s * RB, RB)],
                    acc.at[pl.ds(s * RB, RB)])
    plsc.subcore_barrier()

    e0 = c * EPAD + s * (EPAD // 16)
    d0 = s * (EPAD // 16)

    def chunk(g, _):
        pltpu.sync_copy(srcs_hbm.at[pl.ds(e0 + g * K, K)], sbuf.at[0])
        pltpu.sync_copy(dst_hbm.at[pl.ds(d0 + g * K, K)], dbuf.at[0])
        pltpu.sync_copy(hs_hbm.at[sbuf.at[0]], rows)
        pltpu.sync_copy(rows, acc.at[dbuf.at[0]], add=True)
        return 0

    lax.fori_loop(0, EPAD // 16 // K, chunk, 0)
    plsc.subcore_barrier()
    pltpu.sync_copy(acc.at[pl.ds(s * RB, RB)],
                    out_hbm.at[pl.ds(c * NP + s * RB, RB)])


def _tc1(x_ref, w_ref, deg_ref, out_ref):
    dinv = lax.rsqrt(deg_ref[0, :] + deg_ref[1, :] + 1.0)
    t = jnp.dot(x_ref[...], w_ref[...], preferred_element_type=jnp.float32)
    t = t * dinv[:, None]
    out_ref[0] = t[:, :HH]
    out_ref[1] = t[:, HH:]


def _tc2(a_ref, deg_ref, b_ref, g_ref, be_ref, w_ref, out_ref):
    dinv = lax.rsqrt(deg_ref[0, :] + deg_ref[1, :] + 1.0)
    a = jnp.concatenate([a_ref[0], a_ref[1]], axis=1)
    h = a * dinv[:, None] + b_ref[...]
    h = jnp.maximum(h * (g_ref[...] * BNS) + be_ref[...], 0.0)
    t = jnp.dot(h, w_ref[...], preferred_element_type=jnp.float32)
    t = t * dinv[:, None]
    out_ref[0] = t[:, :HH]
    out_ref[1] = t[:, HH:]


def _tc3(a_ref, deg_ref, b_ref, g_ref, be_ref, w_ref, fb_ref, sec_ref,
         tsum_ref, cnt_ref):
    i = pl.program_id(0)
    dinv = lax.rsqrt(deg_ref[0, :] + deg_ref[1, :] + 1.0)
    a = jnp.concatenate([a_ref[0], a_ref[1]], axis=1)
    h = a * dinv[:, None] + b_ref[...]
    h = jnp.maximum(h * (g_ref[...] * BNS) + be_ref[...], 0.0)
    t = jnp.maximum(
        jnp.dot(h, w_ref[...], preferred_element_type=jnp.float32)
        + fb_ref[...], 0.0)
    iot = lax.broadcasted_iota(jnp.int32, (1, S), 1)
    oh = (sec_ref[...] == iot).astype(jnp.float32)
    ts = lax.dot_general(oh, t, (((0,), (0,)), ((), ())),
                         preferred_element_type=jnp.float32)
    cs = jnp.sum(oh, axis=0)[:, None]

    @pl.when(i == 0)
    def _():
        tsum_ref[...] = ts
        cnt_ref[...] = cs

    @pl.when(i > 0)
    def _():
        tsum_ref[...] += ts
        cnt_ref[...] += cs


def _tc4(ts_ref, cnt_ref, w2_ref, b2_ref, hw1_ref, hb1_ref, hw2_ref,
         hb2_ref, out_ref):
    cnt = cnt_ref[...]
    meant = ts_ref[...] / jnp.maximum(cnt, 1.0)
    se = jnp.dot(meant, w2_ref[...], preferred_element_type=jnp.float32)
    se = se + b2_ref[...]
    se = jnp.where(cnt > 0.0, se, 0.0)
    rows = []
    for k in range(S):
        v = jnp.dot(se[k:k + 1, :], hw1_ref[k],
                    preferred_element_type=jnp.float32) + hb1_ref[k:k + 1, :]
        v = jnp.maximum(v, 0.0)
        p = jnp.sum(v * hw2_ref[k], axis=1, keepdims=True) + hb2_ref[k:k + 1, :]
        rows.append(p)
    out_ref[...] = jnp.concatenate(rows, axis=0)


def kernel(x, edge_index, sectors, W0, b0, W1, b1, g0, be0, g1, be1,
           fcW1, fcb1, fcW2, fcb2, HW1, Hb1, HW2, Hb2):
    f32 = jnp.float32
    src, dst = edge_index[0], edge_index[1]
    padn = NP - N
    x_pad = jnp.pad(x, ((0, padn), (0, 0)))
    sec_pad = jnp.pad(sectors, (0, padn), constant_values=S)[:, None]
    pade = EPAD - E
    filler = N + (jnp.arange(pade, dtype=jnp.int32) % padn)
    src_p = jnp.concatenate([src, filler])
    dst_p = jnp.concatenate([dst, filler])
    srcs2 = jnp.concatenate([src_p, src_p + NP])

    degpair = _deg(dst_p).reshape(2, NP)

    hs0 = pl.pallas_call(
        _tc1, grid=(NB,),
        in_specs=[pl.BlockSpec((R, D_IN), lambda i: (i, 0)),
                  pl.BlockSpec((D_IN, H), lambda i: (0, 0)),
                  pl.BlockSpec((2, R), lambda i: (0, i))],
        out_specs=pl.BlockSpec((2, R, HH), lambda i: (0, i, 0)),
        out_shape=jax.ShapeDtypeStruct((2, NP, HH), f32),
    )(x_pad, W0, degpair)

    agg0 = _agg(hs0.reshape(2 * NP, HH), srcs2, dst_p).reshape(2, NP, HH)

    hs1 = pl.pallas_call(
        _tc2, grid=(NB,),
        in_specs=[pl.BlockSpec((2, R, HH), lambda i: (0, i, 0)),
                  pl.BlockSpec((2, R), lambda i: (0, i)),
                  pl.BlockSpec((1, H), lambda i: (0, 0)),
                  pl.BlockSpec((1, H), lambda i: (0, 0)),
                  pl.BlockSpec((1, H), lambda i: (0, 0)),
                  pl.BlockSpec((H, H), lambda i: (0, 0))],
        out_specs=pl.BlockSpec((2, R, HH), lambda i: (0, i, 0)),
        out_shape=jax.ShapeDtypeStruct((2, NP, HH), f32),
    )(agg0, degpair, b0[None, :], g0[None, :], be0[None, :], W1)

    agg1 = _agg(hs1.reshape(2 * NP, HH), srcs2, dst_p).reshape(2, NP, HH)

    tsum, cnt = pl.pallas_call(
        _tc3, grid=(NB,),
        in_specs=[pl.BlockSpec((2, R, HH), lambda i: (0, i, 0)),
                  pl.BlockSpec((2, R), lambda i: (0, i)),
                  pl.BlockSpec((1, H), lambda i: (0, 0)),
                  pl.BlockSpec((1, H), lambda i: (0, 0)),
                  pl.BlockSpec((1, H), lambda i: (0, 0)),
                  pl.BlockSpec((H, HH), lambda i: (0, 0)),
                  pl.BlockSpec((1, HH), lambda i: (0, 0)),
                  pl.BlockSpec((R, 1), lambda i: (i, 0))],
        out_specs=[pl.BlockSpec((S, HH), lambda i: (0, 0)),
                   pl.BlockSpec((S, 1), lambda i: (0, 0))],
        out_shape=[jax.ShapeDtypeStruct((S, HH), f32),
                   jax.ShapeDtypeStruct((S, 1), f32)],
    )(agg1, degpair, b1[None, :], g1[None, :], be1[None, :], fcW1,
      fcb1[None, :], sec_pad)

    preds = pl.pallas_call(
        _tc4,
        out_shape=jax.ShapeDtypeStruct((S, 1), f32),
    )(tsum, cnt, fcW2, fcb2[None, :], HW1, Hb1,
      jnp.transpose(HW2, (0, 2, 1)), Hb2)
    return preds


# speedup vs baseline: 25.6273x; 2.2567x over previous
import functools

import jax
import jax.numpy as jnp
from jax import lax
from jax.experimental import pallas as pl
from jax.experimental.pallas import tpu as pltpu
from jax.experimental.pallas import tpu_sc as plsc

N = 10000
NP = 10240
E = 320000
EPAD = 327680
D_IN = 128
H = 256
HH = 128
S = 11
EPS = 1e-5
BNS = 1.0 / (1.0 + EPS) ** 0.5
K = 128
RB = NP // 16
R = 1024
NB = NP // R
NG = EPAD // 16 // K
NBUF = 4
NGD = EPAD // 32 // K

_mesh = plsc.VectorSubcoreMesh(core_axis_name="c", subcore_axis_name="s")


@functools.partial(
    pl.kernel, mesh=_mesh,
    out_type=jax.ShapeDtypeStruct((2 * NP,), jnp.float32),
    scratch_types=[
        pltpu.VMEM((NGD, K), jnp.int32),
        pltpu.VMEM((K,), jnp.float32),
        pltpu.VMEM((RB,), jnp.float32),
        pltpu.VMEM_SHARED((NP,), jnp.float32),
    ],
)
def _deg(dst3_hbm, out_hbm, didx, ones_v, zbuf, acc):
    c = lax.axis_index("c")
    s = lax.axis_index("s")
    w = c * 16 + s

    def fill_ones(i, _):
        ones_v[pl.ds(i * 16, 16)] = jnp.ones((16,), jnp.float32)
        return 0

    lax.fori_loop(0, K // 16, fill_ones, 0)

    def fill_zero(i, _):
        zbuf[pl.ds(i * 16, 16)] = jnp.zeros((16,), jnp.float32)
        return 0

    lax.fori_loop(0, RB // 16, fill_zero, 0)
    pltpu.sync_copy(dst3_hbm.at[w], didx)
    pltpu.sync_copy(zbuf, acc.at[pl.ds(s * RB, RB)])
    plsc.subcore_barrier()

    def chunk(g, _):
        pltpu.sync_copy(ones_v, acc.at[didx.at[g]], add=True)
        return 0

    lax.fori_loop(0, NGD, chunk, 0)
    plsc.subcore_barrier()
    pltpu.sync_copy(acc.at[pl.ds(s * RB, RB)],
                    out_hbm.at[pl.ds(c * NP + s * RB, RB)])


@functools.partial(
    pl.kernel, mesh=_mesh,
    out_type=jax.ShapeDtypeStruct((2 * NP, HH), jnp.float32),
    scratch_types=[
        pltpu.VMEM((4, K), jnp.int32),
        pltpu.VMEM((4, K), jnp.int32),
        pltpu.VMEM((K, HH), jnp.float32),
        pltpu.VMEM((K, HH), jnp.float32),
        pltpu.SemaphoreType.DMA,
        pltpu.SemaphoreType.DMA,
        pltpu.SemaphoreType.DMA,
        pltpu.SemaphoreType.DMA,
        pltpu.SemaphoreType.DMA,
        pltpu.SemaphoreType.DMA,
        pltpu.SemaphoreType.DMA,
        pltpu.SemaphoreType.DMA,
        pltpu.SemaphoreType.DMA,
        pltpu.SemaphoreType.DMA,
        pltpu.VMEM_SHARED((NP, HH), jnp.float32),
    ],
)
def _agg(hs_hbm, srcs3_hbm, dst3_hbm, out_hbm, sidx, didx, r0, r1,
         ss0, ss1, ss2, ss3, ds0, ds1, ds2, ds3, gs0, gs1, acc):
    c = lax.axis_index("c")
    s = lax.axis_index("s")
    w = c * 16 + s
    rows = [r0, r1]
    ssem = [ss0, ss1, ss2, ss3]
    dsem = [ds0, ds1, ds2, ds3]
    gsem = [gs0, gs1]

    def idx_start(g, jb):
        pltpu.make_async_copy(srcs3_hbm.at[w].at[g], sidx.at[jb],
                              ssem[jb]).start()
        pltpu.make_async_copy(dst3_hbm.at[s].at[g], didx.at[jb],
                              dsem[jb]).start()

    def idx_wait(g, jb):
        pltpu.make_async_copy(srcs3_hbm.at[w].at[g], sidx.at[jb],
                              ssem[jb]).wait()
        pltpu.make_async_copy(dst3_hbm.at[s].at[g], didx.at[jb],
                              dsem[jb]).wait()

    def gat_start(jb, b):
        pltpu.make_async_copy(hs_hbm.at[sidx.at[jb]], rows[b],
                              gsem[b]).start()

    def gat_wait(jb, b):
        pltpu.make_async_copy(hs_hbm.at[sidx.at[jb]], rows[b],
                              gsem[b]).wait()

    pltpu.sync_copy(hs_hbm.at[pl.ds(c * NP + s * RB, RB)],
                    acc.at[pl.ds(s * RB, RB)])
    for j in range(3):
        idx_start(j, j)
    plsc.subcore_barrier()

    def outer(g0, _):
        for k in range(4):
            g = g0 * 4 + k
            b = k % 2
            idx_wait(g, k)
            gat_start(k, b)
            if k == 0:
                @pl.when(g0 >= 1)
                def _():
                    gat_wait(3, 1 - b)
                    pltpu.sync_copy(rows[1 - b], acc.at[didx.at[3]],
                                    add=True)
                idx_start(g + 3, 3)
            else:
                gat_wait(k - 1, 1 - b)
                pltpu.sync_copy(rows[1 - b], acc.at[didx.at[k - 1]],
                                add=True)

                @pl.when(g0 < NG // 4 - 1)
                def _():
                    idx_start(g + 3, k - 1)
        return 0

    lax.fori_loop(0, NG // 4, outer, 0)
    gat_wait(3, 1)
    pltpu.sync_copy(rows[1], acc.at[didx.at[3]], add=True)
    plsc.subcore_barrier()
    pltpu.sync_copy(acc.at[pl.ds(s * RB, RB)],
                    out_hbm.at[pl.ds(c * NP + s * RB, RB)])


def _tc1(x_ref, w_ref, deg_ref, out_ref):
    dinv = lax.rsqrt(deg_ref[0, :] + deg_ref[1, :] + 1.0)
    t = jnp.dot(x_ref[...], w_ref[...], preferred_element_type=jnp.float32)
    t = t * dinv[:, None]
    out_ref[0] = t[:, :HH]
    out_ref[1] = t[:, HH:]


def _tc2(a_ref, deg_ref, b_ref, g_ref, be_ref, w_ref, out_ref):
    dinv = lax.rsqrt(deg_ref[0, :] + deg_ref[1, :] + 1.0)
    a = jnp.concatenate([a_ref[0], a_ref[1]], axis=1)
    h = a * dinv[:, None] + b_ref[...]
    h = jnp.maximum(h * (g_ref[...] * BNS) + be_ref[...], 0.0)
    t = jnp.dot(h, w_ref[...], preferred_element_type=jnp.float32)
    t = t * dinv[:, None]
    out_ref[0] = t[:, :HH]
    out_ref[1] = t[:, HH:]


def _tc3(a_ref, deg_ref, b_ref, g_ref, be_ref, w_ref, fb_ref, sec_ref,
         tsum_ref, cnt_ref):
    i = pl.program_id(0)
    dinv = lax.rsqrt(deg_ref[0, :] + deg_ref[1, :] + 1.0)
    a = jnp.concatenate([a_ref[0], a_ref[1]], axis=1)
    h = a * dinv[:, None] + b_ref[...]
    h = jnp.maximum(h * (g_ref[...] * BNS) + be_ref[...], 0.0)
    t = jnp.maximum(
        jnp.dot(h, w_ref[...], preferred_element_type=jnp.float32)
        + fb_ref[...], 0.0)
    iot = lax.broadcasted_iota(jnp.int32, (1, S), 1)
    oh = (sec_ref[...] == iot).astype(jnp.float32)
    ts = lax.dot_general(oh, t, (((0,), (0,)), ((), ())),
                         preferred_element_type=jnp.float32)
    cs = jnp.sum(oh, axis=0)[:, None]

    @pl.when(i == 0)
    def _():
        tsum_ref[...] = ts
        cnt_ref[...] = cs

    @pl.when(i > 0)
    def _():
        tsum_ref[...] += ts
        cnt_ref[...] += cs


def _tc4(ts_ref, cnt_ref, w2_ref, b2_ref, hw1_ref, hb1_ref, hw2_ref,
         hb2_ref, out_ref):
    cnt = cnt_ref[...]
    meant = ts_ref[...] / jnp.maximum(cnt, 1.0)
    se = jnp.dot(meant, w2_ref[...], preferred_element_type=jnp.float32)
    se = se + b2_ref[...]
    se = jnp.where(cnt > 0.0, se, 0.0)
    rows = []
    for k in range(S):
        v = jnp.dot(se[k:k + 1, :], hw1_ref[k],
                    preferred_element_type=jnp.float32) + hb1_ref[k:k + 1, :]
        v = jnp.maximum(v, 0.0)
        p = jnp.sum(v * hw2_ref[k], axis=1, keepdims=True) + hb2_ref[k:k + 1, :]
        rows.append(p)
    out_ref[...] = jnp.concatenate(rows, axis=0)


def kernel(x, edge_index, sectors, W0, b0, W1, b1, g0, be0, g1, be1,
           fcW1, fcb1, fcW2, fcb2, HW1, Hb1, HW2, Hb2):
    f32 = jnp.float32
    src, dst = edge_index[0], edge_index[1]
    padn = NP - N
    x_pad = jnp.pad(x, ((0, padn), (0, 0)))
    sec_pad = jnp.pad(sectors, (0, padn), constant_values=S)[:, None]
    pade = EPAD - E
    filler = N + (jnp.arange(pade, dtype=jnp.int32) % padn)
    src_p = jnp.concatenate([src, filler])
    dst_p = jnp.concatenate([dst, filler])
    srcs3 = jnp.concatenate([src_p, src_p + NP]).reshape(32, NG, K)
    dst3 = dst_p.reshape(16, NG, K)
    dst3d = dst_p.reshape(32, NGD, K)

    degpair = _deg(dst3d).reshape(2, NP)

    hs0 = pl.pallas_call(
        _tc1, grid=(NB,),
        in_specs=[pl.BlockSpec((R, D_IN), lambda i: (i, 0)),
                  pl.BlockSpec((D_IN, H), lambda i: (0, 0)),
                  pl.BlockSpec((2, R), lambda i: (0, i))],
        out_specs=pl.BlockSpec((2, R, HH), lambda i: (0, i, 0)),
        out_shape=jax.ShapeDtypeStruct((2, NP, HH), f32),
    )(x_pad, W0, degpair)

    agg0 = _agg(hs0.reshape(2 * NP, HH), srcs3, dst3).reshape(2, NP, HH)

    hs1 = pl.pallas_call(
        _tc2, grid=(NB,),
        in_specs=[pl.BlockSpec((2, R, HH), lambda i: (0, i, 0)),
                  pl.BlockSpec((2, R), lambda i: (0, i)),
                  pl.BlockSpec((1, H), lambda i: (0, 0)),
                  pl.BlockSpec((1, H), lambda i: (0, 0)),
                  pl.BlockSpec((1, H), lambda i: (0, 0)),
                  pl.BlockSpec((H, H), lambda i: (0, 0))],
        out_specs=pl.BlockSpec((2, R, HH), lambda i: (0, i, 0)),
        out_shape=jax.ShapeDtypeStruct((2, NP, HH), f32),
    )(agg0, degpair, b0[None, :], g0[None, :], be0[None, :], W1)

    agg1 = _agg(hs1.reshape(2 * NP, HH), srcs3, dst3).reshape(2, NP, HH)

    tsum, cnt = pl.pallas_call(
        _tc3, grid=(NB,),
        in_specs=[pl.BlockSpec((2, R, HH), lambda i: (0, i, 0)),
                  pl.BlockSpec((2, R), lambda i: (0, i)),
                  pl.BlockSpec((1, H), lambda i: (0, 0)),
                  pl.BlockSpec((1, H), lambda i: (0, 0)),
                  pl.BlockSpec((1, H), lambda i: (0, 0)),
                  pl.BlockSpec((H, HH), lambda i: (0, 0)),
                  pl.BlockSpec((1, HH), lambda i: (0, 0)),
                  pl.BlockSpec((R, 1), lambda i: (i, 0))],
        out_specs=[pl.BlockSpec((S, HH), lambda i: (0, 0)),
                   pl.BlockSpec((S, 1), lambda i: (0, 0))],
        out_shape=[jax.ShapeDtypeStruct((S, HH), f32),
                   jax.ShapeDtypeStruct((S, 1), f32)],
    )(agg1, degpair, b1[None, :], g1[None, :], be1[None, :], fcW1,
      fcb1[None, :], sec_pad)

    preds = pl.pallas_call(
        _tc4,
        out_shape=jax.ShapeDtypeStruct((S, 1), f32),
    )(tsum, cnt, fcW2, fcb2[None, :], HW1, Hb1,
      jnp.transpose(HW2, (0, 2, 1)), Hb2)
    return preds


# speedup vs baseline: 25.7921x; 1.0064x over previous
import functools

import jax
import jax.numpy as jnp
from jax import lax
from jax.experimental import pallas as pl
from jax.experimental.pallas import tpu as pltpu
from jax.experimental.pallas import tpu_sc as plsc

N = 10000
NP = 10240
E = 320000
EPAD = 327680
D_IN = 128
H = 256
HH = 128
S = 11
EPS = 1e-5
BNS = 1.0 / (1.0 + EPS) ** 0.5
K = 128
RB = NP // 16
R = 1024
NB = NP // R
NG = EPAD // 16 // K
NBUF = 4
NGD = EPAD // 32 // K

_mesh = plsc.VectorSubcoreMesh(core_axis_name="c", subcore_axis_name="s")


@functools.partial(
    pl.kernel, mesh=_mesh,
    out_type=jax.ShapeDtypeStruct((2 * NP,), jnp.float32),
    scratch_types=[
        pltpu.VMEM((NGD, K), jnp.int32),
        pltpu.VMEM((K,), jnp.float32),
        pltpu.VMEM((RB,), jnp.float32),
        pltpu.VMEM_SHARED((NP,), jnp.float32),
    ],
)
def _deg(dst3_hbm, out_hbm, didx, ones_v, zbuf, acc):
    c = lax.axis_index("c")
    s = lax.axis_index("s")
    w = c * 16 + s

    def fill_ones(i, _):
        ones_v[pl.ds(i * 16, 16)] = jnp.ones((16,), jnp.float32)
        return 0

    lax.fori_loop(0, K // 16, fill_ones, 0)

    def fill_zero(i, _):
        zbuf[pl.ds(i * 16, 16)] = jnp.zeros((16,), jnp.float32)
        return 0

    lax.fori_loop(0, RB // 16, fill_zero, 0)
    pltpu.sync_copy(dst3_hbm.at[w], didx)
    pltpu.sync_copy(zbuf, acc.at[pl.ds(s * RB, RB)])
    plsc.subcore_barrier()

    def chunk(g, _):
        pltpu.sync_copy(ones_v, acc.at[didx.at[g]], add=True)
        return 0

    lax.fori_loop(0, NGD, chunk, 0)
    plsc.subcore_barrier()
    pltpu.sync_copy(acc.at[pl.ds(s * RB, RB)],
                    out_hbm.at[pl.ds(c * NP + s * RB, RB)])


@functools.partial(
    pl.kernel, mesh=_mesh,
    out_type=jax.ShapeDtypeStruct((2 * NP, HH), jnp.float32),
    scratch_types=[
        pltpu.VMEM((4, K), jnp.int32),
        pltpu.VMEM((4, K), jnp.int32),
        pltpu.VMEM((K, HH), jnp.float32),
        pltpu.VMEM((K, HH), jnp.float32),
        pltpu.SemaphoreType.DMA,
        pltpu.SemaphoreType.DMA,
        pltpu.SemaphoreType.DMA,
        pltpu.SemaphoreType.DMA,
        pltpu.SemaphoreType.DMA,
        pltpu.SemaphoreType.DMA,
        pltpu.SemaphoreType.DMA,
        pltpu.SemaphoreType.DMA,
        pltpu.SemaphoreType.DMA,
        pltpu.SemaphoreType.DMA,
        pltpu.SemaphoreType.DMA,
        pltpu.SemaphoreType.DMA,
        pltpu.VMEM_SHARED((NP, HH), jnp.float32),
    ],
)
def _agg(hs_hbm, srcs3_hbm, dst3_hbm, out_hbm, sidx, didx, r0, r1,
         ss0, ss1, ss2, ss3, ds0, ds1, ds2, ds3, gs0, gs1, cs0, cs1, acc):
    c = lax.axis_index("c")
    s = lax.axis_index("s")
    w = c * 16 + s
    rows = [r0, r1]
    ssem = [ss0, ss1, ss2, ss3]
    dsem = [ds0, ds1, ds2, ds3]
    gsem = [gs0, gs1]
    csem = [cs0, cs1]

    def idx_start(g, jb):
        pltpu.make_async_copy(srcs3_hbm.at[w].at[g], sidx.at[jb],
                              ssem[jb]).start()
        pltpu.make_async_copy(dst3_hbm.at[s].at[g], didx.at[jb],
                              dsem[jb]).start()

    def idx_wait(g, jb):
        pltpu.make_async_copy(srcs3_hbm.at[w].at[g], sidx.at[jb],
                              ssem[jb]).wait()
        pltpu.make_async_copy(dst3_hbm.at[s].at[g], didx.at[jb],
                              dsem[jb]).wait()

    def gat_start(jb, b):
        pltpu.make_async_copy(hs_hbm.at[sidx.at[jb]], rows[b],
                              gsem[b]).start()

    def gat_wait(jb, b):
        pltpu.make_async_copy(hs_hbm.at[sidx.at[jb]], rows[b],
                              gsem[b]).wait()

    def sc_start(jb, b):
        pltpu.async_copy(rows[b], acc.at[didx.at[jb]], csem[b], add=True)

    def sc_wait(jb, b):
        pltpu.make_async_copy(rows[b], acc.at[didx.at[jb]], csem[b]).wait()

    pltpu.sync_copy(hs_hbm.at[pl.ds(c * NP + s * RB, RB)],
                    acc.at[pl.ds(s * RB, RB)])
    idx_start(0, 0)
    idx_start(1, 1)
    plsc.subcore_barrier()

    def outer(g0, _):
        for k in range(4):
            g = g0 * 4 + k
            b = k % 2
            if k >= 2:
                sc_wait((k + 2) % 4, b)
            else:
                @pl.when(g0 >= 1)
                def _():
                    sc_wait((k + 2) % 4, b)
            idx_wait(g, k)
            gat_start(k, b)
            if k >= 1:
                gat_wait(k - 1, 1 - b)
                sc_start(k - 1, 1 - b)
            else:
                @pl.when(g0 >= 1)
                def _():
                    gat_wait(3, 1 - b)
                    sc_start(3, 1 - b)
            if k <= 1:
                idx_start(g + 2, (k + 2) % 4)
            else:
                @pl.when(g0 < NG // 4 - 1)
                def _():
                    idx_start(g + 2, (k + 2) % 4)
        return 0

    lax.fori_loop(0, NG // 4, outer, 0)
    gat_wait(3, 1)
    sc_start(3, 1)
    sc_wait(2, 0)
    sc_wait(3, 1)
    plsc.subcore_barrier()
    pltpu.sync_copy(acc.at[pl.ds(s * RB, RB)],
                    out_hbm.at[pl.ds(c * NP + s * RB, RB)])


def _tc1(x_ref, w_ref, deg_ref, out_ref):
    dinv = lax.rsqrt(deg_ref[0, :] + deg_ref[1, :] + 1.0)
    t = jnp.dot(x_ref[...], w_ref[...], preferred_element_type=jnp.float32)
    t = t * dinv[:, None]
    out_ref[0] = t[:, :HH]
    out_ref[1] = t[:, HH:]


def _tc2(a_ref, deg_ref, b_ref, g_ref, be_ref, w_ref, out_ref):
    dinv = lax.rsqrt(deg_ref[0, :] + deg_ref[1, :] + 1.0)
    a = jnp.concatenate([a_ref[0], a_ref[1]], axis=1)
    h = a * dinv[:, None] + b_ref[...]
    h = jnp.maximum(h * (g_ref[...] * BNS) + be_ref[...], 0.0)
    t = jnp.dot(h, w_ref[...], preferred_element_type=jnp.float32)
    t = t * dinv[:, None]
    out_ref[0] = t[:, :HH]
    out_ref[1] = t[:, HH:]


def _tc3(a_ref, deg_ref, b_ref, g_ref, be_ref, w_ref, fb_ref, sec_ref,
         tsum_ref, cnt_ref):
    i = pl.program_id(0)
    dinv = lax.rsqrt(deg_ref[0, :] + deg_ref[1, :] + 1.0)
    a = jnp.concatenate([a_ref[0], a_ref[1]], axis=1)
    h = a * dinv[:, None] + b_ref[...]
    h = jnp.maximum(h * (g_ref[...] * BNS) + be_ref[...], 0.0)
    t = jnp.maximum(
        jnp.dot(h, w_ref[...], preferred_element_type=jnp.float32)
        + fb_ref[...], 0.0)
    iot = lax.broadcasted_iota(jnp.int32, (1, S), 1)
    oh = (sec_ref[...] == iot).astype(jnp.float32)
    ts = lax.dot_general(oh, t, (((0,), (0,)), ((), ())),
                         preferred_element_type=jnp.float32)
    cs = jnp.sum(oh, axis=0)[:, None]

    @pl.when(i == 0)
    def _():
        tsum_ref[...] = ts
        cnt_ref[...] = cs

    @pl.when(i > 0)
    def _():
        tsum_ref[...] += ts
        cnt_ref[...] += cs


def _tc4(ts_ref, cnt_ref, w2_ref, b2_ref, hw1_ref, hb1_ref, hw2_ref,
         hb2_ref, out_ref):
    cnt = cnt_ref[...]
    meant = ts_ref[...] / jnp.maximum(cnt, 1.0)
    se = jnp.dot(meant, w2_ref[...], preferred_element_type=jnp.float32)
    se = se + b2_ref[...]
    se = jnp.where(cnt > 0.0, se, 0.0)
    rows = []
    for k in range(S):
        v = jnp.dot(se[k:k + 1, :], hw1_ref[k],
                    preferred_element_type=jnp.float32) + hb1_ref[k:k + 1, :]
        v = jnp.maximum(v, 0.0)
        p = jnp.sum(v * hw2_ref[k], axis=1, keepdims=True) + hb2_ref[k:k + 1, :]
        rows.append(p)
    out_ref[...] = jnp.concatenate(rows, axis=0)


def kernel(x, edge_index, sectors, W0, b0, W1, b1, g0, be0, g1, be1,
           fcW1, fcb1, fcW2, fcb2, HW1, Hb1, HW2, Hb2):
    f32 = jnp.float32
    src, dst = edge_index[0], edge_index[1]
    padn = NP - N
    x_pad = jnp.pad(x, ((0, padn), (0, 0)))
    sec_pad = jnp.pad(sectors, (0, padn), constant_values=S)[:, None]
    pade = EPAD - E
    filler = N + (jnp.arange(pade, dtype=jnp.int32) % padn)
    src_p = jnp.concatenate([src, filler])
    dst_p = jnp.concatenate([dst, filler])
    srcs3 = jnp.concatenate([src_p, src_p + NP]).reshape(32, NG, K)
    dst3 = dst_p.reshape(16, NG, K)
    dst3d = dst_p.reshape(32, NGD, K)

    degpair = _deg(dst3d).reshape(2, NP)

    hs0 = pl.pallas_call(
        _tc1, grid=(NB,),
        in_specs=[pl.BlockSpec((R, D_IN), lambda i: (i, 0)),
                  pl.BlockSpec((D_IN, H), lambda i: (0, 0)),
                  pl.BlockSpec((2, R), lambda i: (0, i))],
        out_specs=pl.BlockSpec((2, R, HH), lambda i: (0, i, 0)),
        out_shape=jax.ShapeDtypeStruct((2, NP, HH), f32),
    )(x_pad, W0, degpair)

    agg0 = _agg(hs0.reshape(2 * NP, HH), srcs3, dst3).reshape(2, NP, HH)

    hs1 = pl.pallas_call(
        _tc2, grid=(NB,),
        in_specs=[pl.BlockSpec((2, R, HH), lambda i: (0, i, 0)),
                  pl.BlockSpec((2, R), lambda i: (0, i)),
                  pl.BlockSpec((1, H), lambda i: (0, 0)),
                  pl.BlockSpec((1, H), lambda i: (0, 0)),
                  pl.BlockSpec((1, H), lambda i: (0, 0)),
                  pl.BlockSpec((H, H), lambda i: (0, 0))],
        out_specs=pl.BlockSpec((2, R, HH), lambda i: (0, i, 0)),
        out_shape=jax.ShapeDtypeStruct((2, NP, HH), f32),
    )(agg0, degpair, b0[None, :], g0[None, :], be0[None, :], W1)

    agg1 = _agg(hs1.reshape(2 * NP, HH), srcs3, dst3).reshape(2, NP, HH)

    tsum, cnt = pl.pallas_call(
        _tc3, grid=(NB,),
        in_specs=[pl.BlockSpec((2, R, HH), lambda i: (0, i, 0)),
                  pl.BlockSpec((2, R), lambda i: (0, i)),
                  pl.BlockSpec((1, H), lambda i: (0, 0)),
                  pl.BlockSpec((1, H), lambda i: (0, 0)),
                  pl.BlockSpec((1, H), lambda i: (0, 0)),
                  pl.BlockSpec((H, HH), lambda i: (0, 0)),
                  pl.BlockSpec((1, HH), lambda i: (0, 0)),
                  pl.BlockSpec((R, 1), lambda i: (i, 0))],
        out_specs=[pl.BlockSpec((S, HH), lambda i: (0, 0)),
                   pl.BlockSpec((S, 1), lambda i: (0, 0))],
        out_shape=[jax.ShapeDtypeStruct((S, HH), f32),
                   jax.ShapeDtypeStruct((S, 1), f32)],
    )(agg1, degpair, b1[None, :], g1[None, :], be1[None, :], fcW1,
      fcb1[None, :], sec_pad)

    preds = pl.pallas_call(
        _tc4,
        out_shape=jax.ShapeDtypeStruct((S, 1), f32),
    )(tsum, cnt, fcW2, fcb2[None, :], HW1, Hb1,
      jnp.transpose(HW2, (0, 2, 1)), Hb2)
    return preds
